# Initial kernel scaffold; baseline (speedup 1.0000x reference)
import jax
import jax.numpy as jnp
from jax.experimental import pallas as pl


def kernel(inputs):
    raise NotImplementedError("write your pallas kernel here")


# speedup vs baseline: 30.6370x; 30.6370x over previous
import functools

import jax
import jax.numpy as jnp
from jax import lax
from jax.experimental import pallas as pl
from jax.experimental.pallas import tpu as pltpu
from jax.experimental.pallas import tpu_sc as plsc

B = 16
S = 32768
C = 64
K = 8
L = 16
NW = 32
GROUPS = (B * C) // L
GPW = GROUPS // NW
CPB = C // L
CH = 2048
NCH = S // CH


def _topk_sc(x2):
    mesh = plsc.VectorSubcoreMesh(core_axis_name="c", subcore_axis_name="s")

    @functools.partial(
        pl.kernel,
        mesh=mesh,
        out_type=jax.ShapeDtypeStruct((GROUPS, K, L), jnp.float32),
        compiler_params=pltpu.CompilerParams(use_tc_tiling_on_sc=False),
        scratch_types=[
            pltpu.VMEM((CH, L), jnp.float32),
            pltpu.VMEM((K, L), jnp.float32),
        ],
    )
    def k(x_hbm, out_hbm, buf, top_v):
        wid = lax.axis_index("s") * 2 + lax.axis_index("c")
        for gi in range(GPW):
            g = wid * GPW + gi
            b = g // CPB
            c0 = (g % CPB) * L
            row0 = b * S

            neg = jnp.full((L,), -jnp.inf, jnp.float32)
            regs = (neg,) * K

            def chunk_body(t, regs):
                pltpu.sync_copy(
                    x_hbm.at[pl.ds(row0 + t * CH, CH), pl.ds(c0, L)], buf
                )

                def row_body(i, rs):
                    v = buf[i]
                    out = []
                    for j in range(K):
                        out.append(jnp.maximum(rs[j], v))
                        v = jnp.minimum(rs[j], v)
                    return tuple(out)

                return lax.fori_loop(0, CH, row_body, regs)

            regs = lax.fori_loop(0, NCH, chunk_body, regs)
            for j in range(K):
                top_v[j] = regs[j]
            pltpu.sync_copy(top_v, out_hbm.at[g])

    return k(x2)


def kernel(inputs):
    x2 = inputs.reshape(B * S, C)
    out = _topk_sc(x2)
    out = out.reshape(B, CPB, K, L).transpose(0, 1, 3, 2)
    return out.reshape(B, C * K)


# speedup vs baseline: 36.2080x; 1.1818x over previous
import functools

import jax
import jax.numpy as jnp
from jax import lax
from jax.experimental import pallas as pl
from jax.experimental.pallas import tpu as pltpu
from jax.experimental.pallas import tpu_sc as plsc

B = 16
S = 32768
C = 64
K = 8
L = 16
NW = 32
GROUPS = (B * C) // L
GPW = GROUPS // NW
CPB = C // L
CH = 1024
NCH = S // CH
RB = 8
NB1 = S // RB
NB2 = NB1 // RB
NB3 = NB2 // RB
NCAND = K * RB * L
IB = 128

_MESH = plsc.VectorSubcoreMesh(core_axis_name="c", subcore_axis_name="s")


def _insert8(rs, v):
    out = []
    for j in range(K):
        out.append(jnp.maximum(rs[j], v))
        v = jnp.minimum(rs[j], v)
    return tuple(out)


def _insert8_idx(vs, ids, v, vi):
    nvs, nids = [], []
    for j in range(K):
        c = v > vs[j]
        nvs.append(jnp.where(c, v, vs[j]))
        nids.append(jnp.where(c, vi, ids[j]))
        lo_v = jnp.where(c, vs[j], v)
        lo_i = jnp.where(c, ids[j], vi)
        v, vi = lo_v, lo_i
    return tuple(nvs), tuple(nids)


def _select_blocks(x2):
    @functools.partial(
        pl.kernel,
        mesh=_MESH,
        out_type=jax.ShapeDtypeStruct((GROUPS, K, L), jnp.int32),
        compiler_params=pltpu.CompilerParams(
            use_tc_tiling_on_sc=False, needs_layout_passes=False),
        scratch_types=[
            pltpu.VMEM((CH, L), jnp.float32),
            pltpu.VMEM((CH, L), jnp.float32),
            pltpu.VMEM((NB1, L), jnp.float32),
            pltpu.VMEM((NB2, L), jnp.float32),
            pltpu.VMEM((NB3, L), jnp.float32),
            pltpu.VMEM((K, L), jnp.int32),
            pltpu.SemaphoreType.DMA,
            pltpu.SemaphoreType.DMA,
        ],
    )
    def k(x_hbm, bidx_hbm, buf0, buf1, g1, g2, g3, top_i, sem0, sem1):
        wid = lax.axis_index("s") * 2 + lax.axis_index("c")
        lane = lax.broadcasted_iota(jnp.int32, (L,), 0)
        neg = jnp.full((L,), -jnp.inf, jnp.float32)
        zero = jnp.zeros((L,), jnp.int32)

        def blockmax(src_ref, dst_ref, dst_base, nblk):
            @plsc.parallel_loop(0, nblk, unroll=4)
            def _(ib_):
                base = ib_ * RB
                vs = [src_ref[base + r] for r in range(RB)]
                m01 = jnp.maximum(vs[0], vs[1])
                m23 = jnp.maximum(vs[2], vs[3])
                m45 = jnp.maximum(vs[4], vs[5])
                m67 = jnp.maximum(vs[6], vs[7])
                dst_ref[dst_base + ib_] = jnp.maximum(
                    jnp.maximum(m01, m23), jnp.maximum(m45, m67))

        for gi in range(GPW):
            g = wid * GPW + gi
            b = g // CPB
            c0 = (g % CPB) * L
            row0 = b * S

            def src(t):
                return x_hbm.at[pl.ds(row0 + t * CH, CH), pl.ds(c0, L)]

            pltpu.async_copy(src(0), buf0, sem0)
            pltpu.async_copy(src(1), buf1, sem1)

            def pair(u, _):
                t0 = 2 * u
                pltpu.make_async_copy(src(0), buf0, sem0).wait()
                blockmax(buf0, g1, t0 * (CH // RB), CH // RB)

                @pl.when(u < NCH // 2 - 1)
                def _():
                    pltpu.async_copy(
                        x_hbm.at[pl.ds(row0 + (t0 + 2) * CH, CH),
                                 pl.ds(c0, L)], buf0, sem0)

                pltpu.make_async_copy(src(1), buf1, sem1).wait()
                blockmax(buf1, g1, (t0 + 1) * (CH // RB), CH // RB)

                @pl.when(u < NCH // 2 - 1)
                def _():
                    pltpu.async_copy(
                        x_hbm.at[pl.ds(row0 + (t0 + 3) * CH, CH),
                                 pl.ds(c0, L)], buf1, sem1)

                return 0

            lax.fori_loop(0, NCH // 2, pair, 0)

            blockmax(g1, g2, 0, NB2)
            blockmax(g2, g3, 0, NB3)

            def sel3(i, carry):
                vs, ids = carry
                return _insert8_idx(vs, ids, g3[i],
                                    jnp.full((L,), i, jnp.int32))

            vs, ids = lax.fori_loop(0, NB3, sel3, ((neg,) * K, (zero,) * K))

            for lvl_ref in (g2, g1):
                pids = ids
                vs, ids = (neg,) * K, (zero,) * K
                for j in range(K):
                    base = pids[j] * RB

                    def child(r, carry):
                        cvs, cids = carry
                        row = base + r
                        v = plsc.load_gather(lvl_ref, [row, lane])
                        return _insert8_idx(cvs, cids, v, row)

                    vs, ids = lax.fori_loop(0, RB, child, (vs, ids))

            for j in range(K):
                top_i[j] = ids[j]
            pltpu.sync_copy(top_i, bidx_hbm.at[g])

    return k(x2)


def _refetch_fold(x128, bidx):
    RB2 = RB // 2
    NC2 = K * RB2 * L

    @functools.partial(
        pl.kernel,
        mesh=_MESH,
        out_type=jax.ShapeDtypeStruct((GROUPS, K, L), jnp.float32),
        compiler_params=pltpu.CompilerParams(needs_layout_passes=False),
        scratch_types=[
            pltpu.VMEM((K, L), jnp.int32),
            pltpu.VMEM((NC2 // IB, IB), jnp.int32),
            pltpu.VMEM((NC2, 2 * C), jnp.float32),
            pltpu.VMEM((K, L), jnp.float32),
            pltpu.SemaphoreType.DMA,
            pltpu.SemaphoreType.DMA,
        ],
    )
    def k(x_hbm, bidx_hbm, out_hbm, bidx_v, idxb, cand, top_v, semi, semg):
        wid = lax.axis_index("s") * 2 + lax.axis_index("c")
        lane = lax.broadcasted_iota(jnp.int32, (L,), 0)
        neg = jnp.full((L,), -jnp.inf, jnp.float32)

        for gi in range(GPW):
            g = wid * GPW + gi
            b = g // CPB
            c0 = (g % CPB) * L
            row0 = b * S

            pltpu.async_copy(bidx_hbm.at[g], bidx_v, semi).wait()

            for j in range(K):
                base = (row0 >> 1) + bidx_v[j] * RB2

                def wr(r, _):
                    pos = lane * (K * RB2) + (j * RB2 + r)
                    plsc.store_scatter(
                        idxb, [pos >> 7, pos & (IB - 1)], base + r)
                    return 0

                lax.fori_loop(0, RB2, wr, 0)

            copies = []
            for bi in range(NC2 // IB):
                copies.append(pltpu.async_copy(
                    x_hbm.at[idxb.at[bi]],
                    cand.at[pl.ds(bi * IB, IB)], semg))
            for cp in copies:
                cp.wait()

            def fold(t, rs):
                row = lane * (K * RB2) + t
                v0 = plsc.load_gather(cand, [row, c0 + lane])
                v1 = plsc.load_gather(cand, [row, C + c0 + lane])
                return _insert8(_insert8(rs, v0), v1)

            rs = lax.fori_loop(0, K * RB2, fold, (neg,) * K)

            for j in range(K):
                top_v[j] = rs[j]
            pltpu.sync_copy(top_v, out_hbm.at[g])

    return k(x128, bidx)


def kernel(inputs):
    x2 = inputs.reshape(B * S, C)
    x128 = inputs.reshape(B * S // 2, 2 * C)
    bidx = _select_blocks(x2)
    out = _refetch_fold(x128, bidx)
    out = out.reshape(B, CPB, K, L).transpose(0, 1, 3, 2)
    return out.reshape(B, C * K)
